# Initial kernel scaffold; baseline (speedup 1.0000x reference)
import jax
import jax.numpy as jnp
from jax.experimental import pallas as pl


def kernel(feat, logits_raw, logits_aug, W, b):
    raise NotImplementedError("write your pallas kernel here")


# speedup vs baseline: 3.5178x; 3.5178x over previous
import jax
import jax.numpy as jnp
from jax.experimental import pallas as pl
from jax.experimental.pallas import tpu as pltpu

_B = 128
_D = 1024
_C = 1000
_K = 10
_SCALE = 20.0
_CP = 1024
_CB = 200

_HI = jax.lax.Precision.HIGHEST


def _col_stats(x, n_valid):
    nr, nc = x.shape
    m = jnp.max(x, axis=0, keepdims=True)
    e = jnp.exp(x - m)
    s = jnp.sum(e, axis=0, keepdims=True)
    sx = jnp.sum(e * x, axis=0, keepdims=True)
    ent = m + jnp.log(s) - sx / s
    conf = 1.0 / s
    row = jax.lax.broadcasted_iota(jnp.int32, (nr, nc), 0)
    idx = jnp.min(jnp.where(x == m, row, nr), axis=0, keepdims=True)
    return idx.astype(jnp.float32), ent, conf


def _norm_rows(x):
    n = jnp.sqrt(jnp.sum(x * x, axis=1, keepdims=True))
    return x / jnp.maximum(n, 1e-12)


def _fused_kernel(feat_ref, raw_t_ref, aug_t_ref, wp_ref, bc_ref, out_ref,
                  wn_ref, fn_ref, cent_ref,
                  y_w_ref, k_w_ref, w_w_ref,
                  y_f_ref, k_f_ref, w_f_ref, am_ref):
    b_col = bc_ref[...]
    feat = feat_ref[...]
    W = wp_ref[0:_C, :]

    fn_ref[...] = _norm_rows(feat)
    lt = jax.lax.dot_general(W, feat, (((1,), (1,)), ((), ())),
                             preferred_element_type=jnp.float32) + b_col
    y_f, ent_f, conf_f = _col_stats(lt, _C)
    rmax = jnp.max(raw_t_ref[...], axis=0, keepdims=True)
    amax = jnp.max(aug_t_ref[...], axis=0, keepdims=True)
    rows = jax.lax.broadcasted_iota(jnp.int32, (_C, _B), 0)
    r_idx = jnp.min(jnp.where(raw_t_ref[...] == rmax, rows, _C), axis=0,
                    keepdims=True)
    a_idx = jnp.min(jnp.where(aug_t_ref[...] == amax, rows, _C), axis=0,
                    keepdims=True)
    maskv = (r_idx == a_idx).astype(jnp.float32)
    am_ref[0, 0] = jnp.max(maskv)
    y_f_mod = maskv * y_f + (1.0 - maskv) * float(_C)
    key_f = y_f_mod * 1000.0 + maskv * ent_f
    y_f_ref[0:1, :] = y_f_mod
    k_f_ref[0:1, :] = key_f
    w_f_ref[0:1, :] = jnp.maximum(conf_f, 1e-6) * maskv
    pad = jnp.zeros((7, _B), jnp.float32)
    y_f_ref[1:8, :] = pad - 1.0
    k_f_ref[1:8, :] = pad
    w_f_ref[1:8, :] = pad

    for si in range(8):
        blk = wp_ref[si * _B:(si + 1) * _B, :]
        t = jax.lax.dot_general(W, blk, (((1,), (1,)), ((), ())),
                                preferred_element_type=jnp.float32) + b_col
        y_w, ent_w, conf_w = _col_stats(t, _C)
        real = (jax.lax.broadcasted_iota(jnp.int32, (1, _B), 1)
                + si * _B < _C).astype(jnp.float32)
        y_w_ref[si:si + 1, :] = real * y_w + (1.0 - real) * (-2.0)
        k_w_ref[si:si + 1, :] = real * (y_w * 1000.0 + ent_w)
        w_w_ref[si:si + 1, :] = real * jnp.maximum(conf_w, 1e-6)
        wn_ref[si * _B:(si + 1) * _B, :] = _norm_rows(blk)

    am = am_ref[0, 0]

    kt_w = jnp.transpose(k_w_ref[...])
    yt_w = jnp.transpose(y_w_ref[...])
    kt_f = jnp.transpose(k_f_ref[...])
    yt_f = jnp.transpose(y_f_ref[...])
    lane_col = jax.lax.broadcasted_iota(jnp.int32, (_B, 1), 0)
    lane_row = jax.lax.broadcasted_iota(jnp.int32, (1, _B), 1)

    for si in range(8):
        key_i = k_w_ref[si:si + 1, :]
        y_i = y_w_ref[si:si + 1, :]
        idx_i = lane_row + si * _B
        rank = jnp.zeros((1, _B), jnp.float32)
        for sj in range(8):
            key_j = kt_w[:, sj:sj + 1]
            y_j = yt_w[:, sj:sj + 1]
            idx_j = lane_col + sj * _B
            lt = (key_j < key_i) | ((key_j == key_i) & (idx_j < idx_i))
            rank += jnp.sum(jnp.where((y_j == y_i) & lt, 1.0, 0.0),
                            axis=0, keepdims=True)
        lt = kt_f[:, 0:1] < key_i
        rank += jnp.sum(jnp.where((yt_f[:, 0:1] == y_i) & lt, 1.0, 0.0),
                        axis=0, keepdims=True)
        kept = (rank < float(_K)).astype(jnp.float32)
        kept = am * kept + (1.0 - am)
        w_w_ref[si:si + 1, :] = w_w_ref[si:si + 1, :] * kept

    key_i = k_f_ref[0:1, :]
    y_i = y_f_ref[0:1, :]
    rank = jnp.zeros((1, _B), jnp.float32)
    for sj in range(8):
        lt = kt_w[:, sj:sj + 1] <= key_i
        rank += jnp.sum(jnp.where((yt_w[:, sj:sj + 1] == y_i) & lt, 1.0, 0.0),
                        axis=0, keepdims=True)
    lt = ((kt_f[:, 0:1] < key_i)
          | ((kt_f[:, 0:1] == key_i) & (lane_col < lane_row)))
    rank += jnp.sum(jnp.where((yt_f[:, 0:1] == y_i) & lt, 1.0, 0.0),
                    axis=0, keepdims=True)
    kept_f = am * (rank < float(_K)).astype(jnp.float32)
    w_f_ref[0:1, :] = w_f_ref[0:1, :] * kept_f

    for ci in range(5):
        cls = (jax.lax.broadcasted_iota(jnp.int32, (_CB, 1), 0)
               + ci * _CB).astype(jnp.float32)
        a_f = jnp.where(y_f_ref[0:1, :] == cls, w_f_ref[0:1, :], 0.0)
        acc = jax.lax.dot_general(a_f, fn_ref[...], (((1,), (0,)), ((), ())),
                                  precision=_HI,
                                  preferred_element_type=jnp.float32)
        for sj in range(8):
            a_w = jnp.where(y_w_ref[sj:sj + 1, :] == cls,
                            w_w_ref[sj:sj + 1, :], 0.0)
            acc += jax.lax.dot_general(
                a_w, wn_ref[sj * _B:(sj + 1) * _B, :], (((1,), (0,)), ((), ())),
                precision=_HI, preferred_element_type=jnp.float32)
        cent_ref[ci * _CB:(ci + 1) * _CB, :] = _norm_rows(acc)

    out_ref[...] = _SCALE * jax.lax.dot_general(
        fn_ref[...], cent_ref[...], (((1,), (1,)), ((), ())),
        precision=_HI, preferred_element_type=jnp.float32)


def kernel(feat, logits_raw, logits_aug, W, b):
    w_pad = jnp.pad(W, ((0, _CP - _C), (0, 0)))
    b_col = b.reshape(_C, 1)
    raw_t = logits_raw.T
    aug_t = logits_aug.T
    return pl.pallas_call(
        _fused_kernel,
        out_shape=jax.ShapeDtypeStruct((_B, _C), jnp.float32),
        scratch_shapes=[
            pltpu.VMEM((_CP, _D), jnp.float32),
            pltpu.VMEM((_B, _D), jnp.float32),
            pltpu.VMEM((_C, _D), jnp.float32),
            pltpu.VMEM((8, _B), jnp.float32),
            pltpu.VMEM((8, _B), jnp.float32),
            pltpu.VMEM((8, _B), jnp.float32),
            pltpu.VMEM((8, _B), jnp.float32),
            pltpu.VMEM((8, _B), jnp.float32),
            pltpu.VMEM((8, _B), jnp.float32),
            pltpu.SMEM((1, 1), jnp.float32),
        ],
        compiler_params=pltpu.CompilerParams(
            vmem_limit_bytes=100 * 1024 * 1024,
        ),
    )(feat, raw_t, aug_t, w_pad, b_col)
